# baseline (speedup 1.0000x reference)
import functools

import jax
import jax.numpy as jnp
from jax import lax
from jax.experimental import pallas as pl
from jax.experimental.pallas import tpu as pltpu
from jax.experimental.pallas import tpu_sc as plsc

N = 10000
E = 320000
D = 128
G = 64
NC_OUT = 16

NCORE = 2
NSUB = 16
NW = NCORE * NSUB
NP = 10240
EP = 327680
EPW = EP // NW
CH = 128
NCHUNK = EPW // CH
RPT = NP // NSUB
NPT = NP // NW

_mesh = plsc.VectorSubcoreMesh(core_axis_name="c", subcore_axis_name="s")
_f32 = jnp.float32


def _zero_fill(buf, nrows, ncols):
    def row(i, carry):
        for j in range(ncols // 16):
            buf[i, j * 16:(j + 1) * 16] = jnp.zeros((16,), _f32)
        return carry
    lax.fori_loop(0, nrows, row, 0)


@functools.partial(
    pl.kernel,
    out_type=jax.ShapeDtypeStruct((NCORE, NP, D), _f32),
    mesh=_mesh,
    scratch_types=[
        pltpu.VMEM_SHARED((NP, D), _f32),
        pltpu.VMEM((128, D), _f32),
        pltpu.VMEM((2, 128), jnp.int32),
        pltpu.VMEM((32, D), _f32),
    ],
)
def _deg(dst_hbm, out_hbm, acc_sh, ones, didx, zbuf):
    cid = lax.axis_index("c")
    sid = lax.axis_index("s")
    wid = sid * NCORE + cid
    ebase = wid * EPW

    def orow(i, carry):
        for j in range(D // 16):
            ones[i, j * 16:(j + 1) * 16] = jnp.ones((16,), _f32)
        return carry
    lax.fori_loop(0, 128, orow, 0)
    _zero_fill(zbuf, 32, D)
    for t in range(RPT // 32):
        pltpu.sync_copy(zbuf, acc_sh.at[pl.ds(sid * RPT + t * 32, 32)])
    plsc.subcore_barrier()

    def chunk(c, carry):
        off = ebase + c * 256
        for j in range(2):
            pltpu.sync_copy(dst_hbm.at[pl.ds(off + j * 128, 128)], didx.at[j])
        for j in range(2):
            pltpu.sync_copy(ones, acc_sh.at[didx.at[j]], add=True)
        return carry
    lax.fori_loop(0, EPW // 256, chunk, 0)
    plsc.subcore_barrier()
    for t in range(RPT // 32):
        r0 = sid * RPT + t * 32
        pltpu.sync_copy(acc_sh.at[pl.ds(r0, 32)], zbuf)
        pltpu.sync_copy(zbuf, out_hbm.at[cid, pl.ds(r0, 32)])


IBLK = 16


@functools.partial(
    pl.kernel,
    out_type=jax.ShapeDtypeStruct((NCORE, NP, D), _f32),
    mesh=_mesh,
    scratch_types=[
        pltpu.VMEM_SHARED((NP, D), _f32),
        pltpu.VMEM((2, CH, D), _f32),
        pltpu.VMEM((IBLK, 128), jnp.int32),
        pltpu.VMEM((IBLK, 128), jnp.int32),
        pltpu.VMEM((32, D), _f32),
        pltpu.SemaphoreType.DMA((2,)),
        pltpu.SemaphoreType.DMA((2,)),
    ],
)
def _prop(hs_hbm, src_hbm, dst_hbm, out_hbm, acc_sh, rows, sidx, didx, zbuf,
          gsem, ssem):
    cid = lax.axis_index("c")
    sid = lax.axis_index("s")
    wid = sid * NCORE + cid
    ebase = wid * EPW

    _zero_fill(zbuf, 32, D)
    for t in range(RPT // 32):
        pltpu.sync_copy(zbuf, acc_sh.at[pl.ds(sid * RPT + t * 32, 32)])
    plsc.subcore_barrier()

    erow = wid * (EPW // 128)

    def load_idx_block(b):
        roff = erow + b * IBLK
        pltpu.sync_copy(src_hbm.at[pl.ds(roff, IBLK)], sidx)
        pltpu.sync_copy(dst_hbm.at[pl.ds(roff, IBLK)], didx)

    def start_gather(c, slot):
        pltpu.async_copy(hs_hbm.at[sidx.at[lax.rem(c, IBLK)]],
                         rows.at[slot], gsem.at[slot])

    def wait_gather(c, slot):
        pltpu.make_async_copy(hs_hbm.at[sidx.at[lax.rem(c, IBLK)]],
                              rows.at[slot], gsem.at[slot]).wait()

    def start_scatter(c, slot):
        pltpu.async_copy(rows.at[slot], acc_sh.at[didx.at[lax.rem(c, IBLK)]],
                         ssem.at[slot], add=True)

    def wait_scatter(c, slot):
        pltpu.make_async_copy(rows.at[slot],
                              acc_sh.at[didx.at[lax.rem(c, IBLK)]],
                              ssem.at[slot]).wait()

    load_idx_block(0)
    start_gather(0, 0)

    def chunk(c, carry):
        slot = lax.rem(c, 2)
        nslot = 1 - slot
        wait_gather(c, slot)
        start_scatter(c, slot)

        @pl.when(jnp.logical_and(lax.rem(c + 1, IBLK) == 0, c + 1 < NCHUNK))
        def _():
            load_idx_block((c + 1) // IBLK)

        @pl.when(c + 1 < NCHUNK)
        def _():
            @pl.when(c >= 1)
            def _():
                wait_scatter(c - 1, nslot)
            start_gather(c + 1, nslot)
        return carry
    lax.fori_loop(0, NCHUNK, chunk, 0)
    wait_scatter(NCHUNK - 2, lax.rem(NCHUNK - 2, 2))
    wait_scatter(NCHUNK - 1, lax.rem(NCHUNK - 1, 2))

    plsc.subcore_barrier()
    for t in range(RPT // 32):
        r0 = sid * RPT + t * 32
        pltpu.sync_copy(acc_sh.at[pl.ds(r0, 32)], zbuf)
        pltpu.sync_copy(zbuf, out_hbm.at[cid, pl.ds(r0, 32)])


@functools.partial(
    pl.kernel,
    out_type=(
        jax.ShapeDtypeStruct((NW, G, D), _f32),
        jax.ShapeDtypeStruct((NW, G, D), _f32),
        jax.ShapeDtypeStruct((NW, G, 16), _f32),
    ),
    mesh=_mesh,
    scratch_types=[
        pltpu.VMEM((NPT, D), _f32),
        pltpu.VMEM((NPT,), jnp.int32),
        pltpu.VMEM((G + 1, D), _f32),
        pltpu.VMEM((G + 1, D), _f32),
        pltpu.VMEM((G + 1, 16), _f32),
    ],
)
def _pool(h_hbm, batch_hbm, sum_hbm, max_hbm, cnt_hbm, hbuf, bbuf, sums,
          maxs, cnts):
    cid = lax.axis_index("c")
    sid = lax.axis_index("s")
    wid = sid * NCORE + cid
    nbase = wid * NPT

    _zero_fill(sums, G + 1, D)
    _zero_fill(cnts, G + 1, 16)

    def mrow(i, carry):
        for j in range(D // 16):
            maxs[i, j * 16:(j + 1) * 16] = jnp.full((16,), -jnp.inf, _f32)
        return carry
    lax.fori_loop(0, G + 1, mrow, 0)

    pltpu.sync_copy(batch_hbm.at[pl.ds(nbase, NPT)], bbuf)
    pltpu.sync_copy(h_hbm.at[pl.ds(nbase, NPT)], hbuf)

    def group(i, carry):
        b16 = bbuf[pl.ds(i * 16, 16)]
        for l in range(16):
            g = b16[l]
            n = i * 16 + l
            for j in range(D // 16):
                sl = pl.ds(j * 16, 16)
                v = hbuf[n, sl]
                sums[g, sl] = sums[g, sl] + v
                maxs[g, sl] = jnp.maximum(maxs[g, sl], v)
            cnts[g, :] = cnts[g, :] + jnp.ones((16,), _f32)
        return carry
    lax.fori_loop(0, NPT // 16, group, 0)

    pltpu.sync_copy(sums.at[pl.ds(0, G)], sum_hbm.at[wid])
    pltpu.sync_copy(maxs.at[pl.ds(0, G)], max_hbm.at[wid])
    pltpu.sync_copy(cnts.at[pl.ds(0, G)], cnt_hbm.at[wid])


_R = 2048


def _prep_body(degp_ref, x_ref, w_ref, dis_ref, hs_ref, out_ref):
    deg = degp_ref[0, :, 0:1] + degp_ref[1, :, 0:1]
    dis = jnp.where(deg > 0, lax.rsqrt(jnp.maximum(deg, 1e-30)), 0.0)
    x = x_ref[...]
    dis_ref[...] = dis
    hs_ref[...] = dis * x
    out_ref[...] = jnp.dot(x, w_ref[...], preferred_element_type=_f32)


_prep = pl.pallas_call(
    _prep_body,
    grid=(NP // _R,),
    in_specs=[
        pl.BlockSpec((NCORE, _R, D), lambda i: (0, i, 0)),
        pl.BlockSpec((_R, D), lambda i: (i, 0)),
        pl.BlockSpec((D, D), lambda i: (0, 0)),
    ],
    out_specs=[
        pl.BlockSpec((_R, 1), lambda i: (i, 0)),
        pl.BlockSpec((_R, D), lambda i: (i, 0)),
        pl.BlockSpec((_R, D), lambda i: (i, 0)),
    ],
    out_shape=[
        jax.ShapeDtypeStruct((NP, 1), _f32),
        jax.ShapeDtypeStruct((NP, D), _f32),
        jax.ShapeDtypeStruct((NP, D), _f32),
    ],
)


def _hop_mid_body(p_ref, dis_ref, w_ref, oin_ref, out_ref, hs_ref):
    dis = dis_ref[...]
    h = dis * (p_ref[0] + p_ref[1])
    out_ref[...] = oin_ref[...] + jnp.dot(h, w_ref[...],
                                          preferred_element_type=_f32)
    hs_ref[...] = dis * h


_hop_mid = pl.pallas_call(
    _hop_mid_body,
    grid=(NP // _R,),
    in_specs=[
        pl.BlockSpec((NCORE, _R, D), lambda i: (0, i, 0)),
        pl.BlockSpec((_R, 1), lambda i: (i, 0)),
        pl.BlockSpec((D, D), lambda i: (0, 0)),
        pl.BlockSpec((_R, D), lambda i: (i, 0)),
    ],
    out_specs=[
        pl.BlockSpec((_R, D), lambda i: (i, 0)),
        pl.BlockSpec((_R, D), lambda i: (i, 0)),
    ],
    out_shape=[
        jax.ShapeDtypeStruct((NP, D), _f32),
        jax.ShapeDtypeStruct((NP, D), _f32),
    ],
)


def _fin1_body(p_ref, dis_ref, w_ref, oin_ref, b_ref, w2_ref, out2_ref,
               hs_ref):
    dis = dis_ref[...]
    h3 = dis * (p_ref[0] + p_ref[1])
    out = oin_ref[...] + jnp.dot(h3, w_ref[...],
                                 preferred_element_type=_f32) + b_ref[...]
    h = jnp.maximum(out, 0.0)
    out2_ref[...] = jnp.dot(h, w2_ref[...], preferred_element_type=_f32)
    hs_ref[...] = dis * h


_fin1 = pl.pallas_call(
    _fin1_body,
    grid=(NP // _R,),
    in_specs=[
        pl.BlockSpec((NCORE, _R, D), lambda i: (0, i, 0)),
        pl.BlockSpec((_R, 1), lambda i: (i, 0)),
        pl.BlockSpec((D, D), lambda i: (0, 0)),
        pl.BlockSpec((_R, D), lambda i: (i, 0)),
        pl.BlockSpec((1, D), lambda i: (0, 0)),
        pl.BlockSpec((D, D), lambda i: (0, 0)),
    ],
    out_specs=[
        pl.BlockSpec((_R, D), lambda i: (i, 0)),
        pl.BlockSpec((_R, D), lambda i: (i, 0)),
    ],
    out_shape=[
        jax.ShapeDtypeStruct((NP, D), _f32),
        jax.ShapeDtypeStruct((NP, D), _f32),
    ],
)


def _fin2_body(p_ref, dis_ref, w_ref, oin_ref, b_ref, h_ref):
    h3 = dis_ref[...] * (p_ref[0] + p_ref[1])
    h_ref[...] = oin_ref[...] + jnp.dot(h3, w_ref[...],
                                        preferred_element_type=_f32) + b_ref[...]


_fin2 = pl.pallas_call(
    _fin2_body,
    grid=(NP // _R,),
    in_specs=[
        pl.BlockSpec((NCORE, _R, D), lambda i: (0, i, 0)),
        pl.BlockSpec((_R, 1), lambda i: (i, 0)),
        pl.BlockSpec((D, D), lambda i: (0, 0)),
        pl.BlockSpec((_R, D), lambda i: (i, 0)),
        pl.BlockSpec((1, D), lambda i: (0, 0)),
    ],
    out_specs=pl.BlockSpec((_R, D), lambda i: (i, 0)),
    out_shape=jax.ShapeDtypeStruct((NP, D), _f32),
)


def _final_body(sum_ref, max_ref, cnt_ref, wm_ref, bm_ref, wp_ref, bp_ref,
                y_ref):
    s = jnp.sum(sum_ref[...], axis=0)
    m = jnp.max(max_ref[...], axis=0)
    m = jnp.where(jnp.isfinite(m), m, 0.0)
    c = jnp.sum(cnt_ref[:, :, 0], axis=0)
    mean = s / jnp.maximum(c, 1.0)[:, None]
    g = jnp.maximum(jnp.concatenate([mean, m], axis=1), 0.0)
    gm = jnp.maximum(jnp.dot(g, wm_ref[...],
                             preferred_element_type=_f32) + bm_ref[...], 0.0)
    y_ref[...] = jnp.dot(gm, wp_ref[...],
                         preferred_element_type=_f32) + bp_ref[...]


_final = pl.pallas_call(
    _final_body,
    out_shape=jax.ShapeDtypeStruct((G, NC_OUT), _f32),
)


def kernel(x, edge_index, batch, W1, b1, W2, b2, Wm, bm, Wp, bp):
    src = jnp.concatenate(
        [edge_index[0].astype(jnp.int32),
         jnp.full((EP - E,), NP - 1, jnp.int32)])
    dst = jnp.concatenate(
        [edge_index[1].astype(jnp.int32),
         jnp.full((EP - E,), NP - 1, jnp.int32)])
    x_pad = jnp.pad(x, ((0, NP - N), (0, 0)))
    batch_pad = jnp.concatenate(
        [batch.astype(jnp.int32), jnp.full((NP - N,), G, jnp.int32)])
    b1r, b2r = b1.reshape(1, D), b2.reshape(1, D)
    src2 = src.reshape(EP // 128, 128)
    dst2 = dst.reshape(EP // 128, 128)

    degp = _deg(dst)
    dis, hs, out = _prep(degp, x_pad, W1[0])
    for k in (1, 2):
        out, hs = _hop_mid(_prop(hs, src2, dst2), dis, W1[k], out)
    out, hs = _fin1(_prop(hs, src2, dst2), dis, W1[3], out, b1r, W2[0])
    for k in (1, 2):
        out, hs = _hop_mid(_prop(hs, src2, dst2), dis, W2[k], out)
    h_fin = _fin2(_prop(hs, src2, dst2), dis, W2[3], out, b2r)

    sums, maxs, cnts = _pool(h_fin, batch_pad)
    return _final(sums, maxs, cnts, Wm, bm.reshape(1, D), Wp,
                  bp.reshape(1, NC_OUT))


# speedup vs baseline: 1.0443x; 1.0443x over previous
import functools

import jax
import jax.numpy as jnp
from jax import lax
from jax.experimental import pallas as pl
from jax.experimental.pallas import tpu as pltpu
from jax.experimental.pallas import tpu_sc as plsc

N = 10000
E = 320000
D = 128
G = 64
NC_OUT = 16

NCORE = 2
NSUB = 16
NW = NCORE * NSUB
NP = 10240
EP = 327680
EPW = EP // NW
CH = 128
NCHUNK = EPW // CH
RPT = NP // NSUB
NPT = NP // NW

_mesh = plsc.VectorSubcoreMesh(core_axis_name="c", subcore_axis_name="s")
_f32 = jnp.float32


def _zero_fill(buf, nrows, ncols):
    def row(i, carry):
        for j in range(ncols // 16):
            buf[i, j * 16:(j + 1) * 16] = jnp.zeros((16,), _f32)
        return carry
    lax.fori_loop(0, nrows, row, 0)


@functools.partial(
    pl.kernel,
    out_type=jax.ShapeDtypeStruct((NCORE, NP, D), _f32),
    mesh=_mesh,
    scratch_types=[
        pltpu.VMEM_SHARED((NP, D), _f32),
        pltpu.VMEM((128, D), _f32),
        pltpu.VMEM((2, 128), jnp.int32),
        pltpu.VMEM((32, D), _f32),
    ],
)
def _deg(dst_hbm, out_hbm, acc_sh, ones, didx, zbuf):
    cid = lax.axis_index("c")
    sid = lax.axis_index("s")
    wid = sid * NCORE + cid
    ebase = wid * EPW

    def orow(i, carry):
        for j in range(D // 16):
            ones[i, j * 16:(j + 1) * 16] = jnp.ones((16,), _f32)
        return carry
    lax.fori_loop(0, 128, orow, 0)
    _zero_fill(zbuf, 32, D)
    for t in range(RPT // 32):
        pltpu.sync_copy(zbuf, acc_sh.at[pl.ds(sid * RPT + t * 32, 32)])
    plsc.subcore_barrier()

    def chunk(c, carry):
        off = ebase + c * 256
        for j in range(2):
            pltpu.sync_copy(dst_hbm.at[pl.ds(off + j * 128, 128)], didx.at[j])
        for j in range(2):
            pltpu.sync_copy(ones, acc_sh.at[didx.at[j]], add=True)
        return carry
    lax.fori_loop(0, EPW // 256, chunk, 0)
    plsc.subcore_barrier()
    for t in range(RPT // 32):
        r0 = sid * RPT + t * 32
        pltpu.sync_copy(acc_sh.at[pl.ds(r0, 32)], zbuf)
        pltpu.sync_copy(zbuf, out_hbm.at[cid, pl.ds(r0, 32)])


@functools.partial(
    pl.kernel,
    out_type=jax.ShapeDtypeStruct((NCORE, NP, D), _f32),
    mesh=_mesh,
    scratch_types=[
        pltpu.VMEM_SHARED((NP, D), _f32),
        pltpu.VMEM((2, CH, D), _f32),
        pltpu.VMEM((2, 128), jnp.int32),
        pltpu.VMEM((2, 128), jnp.int32),
        pltpu.VMEM((32, D), _f32),
        pltpu.SemaphoreType.DMA((2,)),
    ],
)
def _prop(hs_hbm, src_hbm, dst_hbm, out_hbm, acc_sh, rows, sidx, didx, zbuf,
          gsem):
    cid = lax.axis_index("c")
    sid = lax.axis_index("s")
    wid = sid * NCORE + cid
    ebase = wid * EPW

    _zero_fill(zbuf, 32, D)
    for t in range(RPT // 32):
        pltpu.sync_copy(zbuf, acc_sh.at[pl.ds(sid * RPT + t * 32, 32)])
    plsc.subcore_barrier()

    def load_idx(c, slot):
        off = ebase + c * CH
        pltpu.sync_copy(src_hbm.at[pl.ds(off, 128)], sidx.at[slot])
        pltpu.sync_copy(dst_hbm.at[pl.ds(off, 128)], didx.at[slot])

    def start_gather(slot):
        pltpu.async_copy(hs_hbm.at[sidx.at[slot]], rows.at[slot],
                         gsem.at[slot])

    def wait_gather(slot):
        pltpu.make_async_copy(hs_hbm.at[sidx.at[slot]], rows.at[slot],
                              gsem.at[slot]).wait()

    def scatter(slot):
        pltpu.sync_copy(rows.at[slot], acc_sh.at[didx.at[slot]], add=True)

    load_idx(0, 0)
    start_gather(0)

    def chunk(c, carry):
        slot = lax.rem(c, 2)
        nslot = 1 - slot

        @pl.when(c < NCHUNK - 1)
        def _():
            load_idx(c + 1, nslot)
            start_gather(nslot)

        wait_gather(slot)
        scatter(slot)
        return carry
    lax.fori_loop(0, NCHUNK, chunk, 0)

    plsc.subcore_barrier()
    for t in range(RPT // 32):
        r0 = sid * RPT + t * 32
        pltpu.sync_copy(acc_sh.at[pl.ds(r0, 32)], zbuf)
        pltpu.sync_copy(zbuf, out_hbm.at[cid, pl.ds(r0, 32)])


@functools.partial(
    pl.kernel,
    out_type=(
        jax.ShapeDtypeStruct((NW, G, D), _f32),
        jax.ShapeDtypeStruct((NW, G, D), _f32),
        jax.ShapeDtypeStruct((NW, G, 16), _f32),
    ),
    mesh=_mesh,
    scratch_types=[
        pltpu.VMEM((NPT, D), _f32),
        pltpu.VMEM((NPT,), jnp.int32),
        pltpu.VMEM((G + 1, D), _f32),
        pltpu.VMEM((G + 1, D), _f32),
        pltpu.VMEM((G + 1, 16), _f32),
    ],
)
def _pool(h_hbm, batch_hbm, sum_hbm, max_hbm, cnt_hbm, hbuf, bbuf, sums,
          maxs, cnts):
    cid = lax.axis_index("c")
    sid = lax.axis_index("s")
    wid = sid * NCORE + cid
    nbase = wid * NPT

    _zero_fill(sums, G + 1, D)
    _zero_fill(cnts, G + 1, 16)

    def mrow(i, carry):
        for j in range(D // 16):
            maxs[i, j * 16:(j + 1) * 16] = jnp.full((16,), -jnp.inf, _f32)
        return carry
    lax.fori_loop(0, G + 1, mrow, 0)

    pltpu.sync_copy(batch_hbm.at[pl.ds(nbase, NPT)], bbuf)
    pltpu.sync_copy(h_hbm.at[pl.ds(nbase, NPT)], hbuf)

    def group(i, carry):
        b16 = bbuf[pl.ds(i * 16, 16)]
        for l in range(16):
            g = b16[l]
            n = i * 16 + l
            for j in range(D // 16):
                sl = pl.ds(j * 16, 16)
                v = hbuf[n, sl]
                sums[g, sl] = sums[g, sl] + v
                maxs[g, sl] = jnp.maximum(maxs[g, sl], v)
            cnts[g, :] = cnts[g, :] + jnp.ones((16,), _f32)
        return carry
    lax.fori_loop(0, NPT // 16, group, 0)

    pltpu.sync_copy(sums.at[pl.ds(0, G)], sum_hbm.at[wid])
    pltpu.sync_copy(maxs.at[pl.ds(0, G)], max_hbm.at[wid])
    pltpu.sync_copy(cnts.at[pl.ds(0, G)], cnt_hbm.at[wid])


_R = 2048


def _prep_body(degp_ref, x_ref, w_ref, dis_ref, hs_ref, out_ref):
    deg = degp_ref[0, :, 0:1] + degp_ref[1, :, 0:1]
    dis = jnp.where(deg > 0, lax.rsqrt(jnp.maximum(deg, 1e-30)), 0.0)
    x = x_ref[...]
    dis_ref[...] = dis
    hs_ref[...] = dis * x
    out_ref[...] = jnp.dot(x, w_ref[...], preferred_element_type=_f32)


_prep = pl.pallas_call(
    _prep_body,
    grid=(NP // _R,),
    in_specs=[
        pl.BlockSpec((NCORE, _R, D), lambda i: (0, i, 0)),
        pl.BlockSpec((_R, D), lambda i: (i, 0)),
        pl.BlockSpec((D, D), lambda i: (0, 0)),
    ],
    out_specs=[
        pl.BlockSpec((_R, 1), lambda i: (i, 0)),
        pl.BlockSpec((_R, D), lambda i: (i, 0)),
        pl.BlockSpec((_R, D), lambda i: (i, 0)),
    ],
    out_shape=[
        jax.ShapeDtypeStruct((NP, 1), _f32),
        jax.ShapeDtypeStruct((NP, D), _f32),
        jax.ShapeDtypeStruct((NP, D), _f32),
    ],
)


def _hop_mid_body(p_ref, dis_ref, w_ref, oin_ref, out_ref, hs_ref):
    dis = dis_ref[...]
    h = dis * (p_ref[0] + p_ref[1])
    out_ref[...] = oin_ref[...] + jnp.dot(h, w_ref[...],
                                          preferred_element_type=_f32)
    hs_ref[...] = dis * h


_hop_mid = pl.pallas_call(
    _hop_mid_body,
    grid=(NP // _R,),
    in_specs=[
        pl.BlockSpec((NCORE, _R, D), lambda i: (0, i, 0)),
        pl.BlockSpec((_R, 1), lambda i: (i, 0)),
        pl.BlockSpec((D, D), lambda i: (0, 0)),
        pl.BlockSpec((_R, D), lambda i: (i, 0)),
    ],
    out_specs=[
        pl.BlockSpec((_R, D), lambda i: (i, 0)),
        pl.BlockSpec((_R, D), lambda i: (i, 0)),
    ],
    out_shape=[
        jax.ShapeDtypeStruct((NP, D), _f32),
        jax.ShapeDtypeStruct((NP, D), _f32),
    ],
)


def _fin1_body(p_ref, dis_ref, w_ref, oin_ref, b_ref, w2_ref, out2_ref,
               hs_ref):
    dis = dis_ref[...]
    h3 = dis * (p_ref[0] + p_ref[1])
    out = oin_ref[...] + jnp.dot(h3, w_ref[...],
                                 preferred_element_type=_f32) + b_ref[...]
    h = jnp.maximum(out, 0.0)
    out2_ref[...] = jnp.dot(h, w2_ref[...], preferred_element_type=_f32)
    hs_ref[...] = dis * h


_fin1 = pl.pallas_call(
    _fin1_body,
    grid=(NP // _R,),
    in_specs=[
        pl.BlockSpec((NCORE, _R, D), lambda i: (0, i, 0)),
        pl.BlockSpec((_R, 1), lambda i: (i, 0)),
        pl.BlockSpec((D, D), lambda i: (0, 0)),
        pl.BlockSpec((_R, D), lambda i: (i, 0)),
        pl.BlockSpec((1, D), lambda i: (0, 0)),
        pl.BlockSpec((D, D), lambda i: (0, 0)),
    ],
    out_specs=[
        pl.BlockSpec((_R, D), lambda i: (i, 0)),
        pl.BlockSpec((_R, D), lambda i: (i, 0)),
    ],
    out_shape=[
        jax.ShapeDtypeStruct((NP, D), _f32),
        jax.ShapeDtypeStruct((NP, D), _f32),
    ],
)


def _fin2_body(p_ref, dis_ref, w_ref, oin_ref, b_ref, h_ref):
    h3 = dis_ref[...] * (p_ref[0] + p_ref[1])
    h_ref[...] = oin_ref[...] + jnp.dot(h3, w_ref[...],
                                        preferred_element_type=_f32) + b_ref[...]


_fin2 = pl.pallas_call(
    _fin2_body,
    grid=(NP // _R,),
    in_specs=[
        pl.BlockSpec((NCORE, _R, D), lambda i: (0, i, 0)),
        pl.BlockSpec((_R, 1), lambda i: (i, 0)),
        pl.BlockSpec((D, D), lambda i: (0, 0)),
        pl.BlockSpec((_R, D), lambda i: (i, 0)),
        pl.BlockSpec((1, D), lambda i: (0, 0)),
    ],
    out_specs=pl.BlockSpec((_R, D), lambda i: (i, 0)),
    out_shape=jax.ShapeDtypeStruct((NP, D), _f32),
)


def _final_body(sum_ref, max_ref, cnt_ref, wm_ref, bm_ref, wp_ref, bp_ref,
                y_ref):
    s = jnp.sum(sum_ref[...], axis=0)
    m = jnp.max(max_ref[...], axis=0)
    m = jnp.where(jnp.isfinite(m), m, 0.0)
    c = jnp.sum(cnt_ref[:, :, 0], axis=0)
    mean = s / jnp.maximum(c, 1.0)[:, None]
    g = jnp.maximum(jnp.concatenate([mean, m], axis=1), 0.0)
    gm = jnp.maximum(jnp.dot(g, wm_ref[...],
                             preferred_element_type=_f32) + bm_ref[...], 0.0)
    y_ref[...] = jnp.dot(gm, wp_ref[...],
                         preferred_element_type=_f32) + bp_ref[...]


_final = pl.pallas_call(
    _final_body,
    out_shape=jax.ShapeDtypeStruct((G, NC_OUT), _f32),
)


def kernel(x, edge_index, batch, W1, b1, W2, b2, Wm, bm, Wp, bp):
    src = jnp.concatenate(
        [edge_index[0].astype(jnp.int32),
         jnp.full((EP - E,), NP - 1, jnp.int32)])
    dst = jnp.concatenate(
        [edge_index[1].astype(jnp.int32),
         jnp.full((EP - E,), NP - 1, jnp.int32)])
    x_pad = jnp.pad(x, ((0, NP - N), (0, 0)))
    batch_pad = jnp.concatenate(
        [batch.astype(jnp.int32), jnp.full((NP - N,), G, jnp.int32)])
    b1r, b2r = b1.reshape(1, D), b2.reshape(1, D)

    degp = _deg(dst)
    dis, hs, out = _prep(degp, x_pad, W1[0])
    for k in (1, 2):
        out, hs = _hop_mid(_prop(hs, src, dst), dis, W1[k], out)
    out, hs = _fin1(_prop(hs, src, dst), dis, W1[3], out, b1r, W2[0])
    for k in (1, 2):
        out, hs = _hop_mid(_prop(hs, src, dst), dis, W2[k], out)
    h_fin = _fin2(_prop(hs, src, dst), dis, W2[3], out, b2r)

    sums, maxs, cnts = _pool(h_fin, batch_pad)
    return _final(sums, maxs, cnts, Wm, bm.reshape(1, D), Wp,
                  bp.reshape(1, NC_OUT))
